# baseline (speedup 1.0000x reference)
import functools

import jax
import jax.numpy as jnp
from jax import lax
from jax.experimental import pallas as pl
from jax.experimental.pallas import tpu as pltpu
from jax.experimental.pallas import tpu_sc as plsc

K = 256
N = 32768
NV = N // 16
ROWS = 64
NWORK = 32
RPW = ROWS // NWORK


def _iota16():
    return lax.broadcasted_iota(jnp.int32, (16,), 0)


def _ukey(x):
    b = lax.bitcast_convert_type(x, jnp.uint32)
    neg = b >> jnp.uint32(31)
    flip = jnp.where(neg == jnp.uint32(1), jnp.uint32(0xFFFFFFFF),
                     jnp.uint32(0x80000000))
    return b ^ flip


def _pcount(m):
    return jnp.max(plsc.all_reduce_population_count(m))


def _find_bucket(tot_ref, needed, iota):
    def body(gi, carry):
        found, b_star, n_gt, above = carry
        g = 15 - gi
        t = tot_ref[pl.ds(g * 16, 16)]
        r = lax.rev(t, (0,))
        c = plsc.cumsum(r)
        s_ge = c + above
        l_star = _pcount(s_ge < needed)
        in_group = l_star < 16
        bs_g = g * 16 + 15 - l_star
        cprev = jnp.sum(jnp.where(iota == (l_star - 1), c, 0))
        s_gt = above + cprev
        take = jnp.logical_and(found == 0, in_group)
        return (jnp.where(take, 1, found),
                jnp.where(take, bs_g, b_star),
                jnp.where(take, s_gt, n_gt),
                above + jnp.sum(t))

    _, b_star, n_gt, _ = lax.fori_loop(0, 16, body, (0, 0, 0, 0))
    return b_star, n_gt


def _totals(hist_ref, tot_ref):
    def body(g, _):
        acc = hist_ref[pl.ds(g * 16, 16)]
        for lane in range(1, 16):
            acc = acc + hist_ref[pl.ds(lane * 256 + g * 16, 16)]
        tot_ref[pl.ds(g * 16, 16)] = acc
        return 0

    lax.fori_loop(0, 16, body, 0)


def _zero_hist(hist_ref):
    z = jnp.zeros((16,), jnp.int32)

    def body(i, _):
        hist_ref[pl.ds(i * 16, 16)] = z
        return 0

    lax.fori_loop(0, 256, body, 0)


def _perm16(x, perm):
    dnums = lax.GatherDimensionNumbers(
        offset_dims=(), collapsed_slice_dims=(0,), start_index_map=(0,))
    return lax.gather(x, perm.reshape(16, 1), dnums, (1,),
                      mode=lax.GatherScatterMode.PROMISE_IN_BOUNDS,
                      unique_indices=True)


def _cmp_gt(ka, ia, kb, ib):
    return (ka > kb) | ((ka == kb) & (ia < ib))


def _bitonic256(ks, js, iota):
    for kk in (2, 4, 8, 16, 32, 64, 128, 256):
        j = kk // 2
        while j >= 1:
            if j >= 16:
                jj = j // 16
                for r in range(16):
                    if r & jj:
                        continue
                    a, b = r, r | jj
                    desc = ((r * 16) & kk) == 0
                    g = _cmp_gt(ks[a], js[a], ks[b], js[b])
                    if desc:
                        sel = g
                    else:
                        sel = jnp.logical_not(g)
                    nka = jnp.where(sel, ks[a], ks[b])
                    nja = jnp.where(sel, js[a], js[b])
                    nkb = jnp.where(sel, ks[b], ks[a])
                    njb = jnp.where(sel, js[b], js[a])
                    ks[a], js[a], ks[b], js[b] = nka, nja, nkb, njb
            else:
                perm = iota ^ j
                is_lo = (iota & j) == 0
                for r in range(16):
                    if kk >= 32 or kk == 16:
                        desc_static = ((r * 16) & kk) == 0
                        want_greater = is_lo if desc_static else (~is_lo)
                    else:
                        desc = ((r * 16 + iota) & kk) == 0
                        want_greater = jnp.logical_not(jnp.logical_xor(desc, is_lo))
                    kp = _perm16(ks[r], perm)
                    ip = _perm16(js[r], perm)
                    g = _cmp_gt(ks[r], js[r], kp, ip)
                    choose_self = want_greater == g
                    ks[r] = jnp.where(choose_self, ks[r], kp)
                    js[r] = jnp.where(choose_self, js[r], ip)
            j //= 2
    return ks, js


def _body(scores_hbm, vals_hbm, idx_hbm, row_v, mid_v, hist_v, tot_v, fin_v,
          ov_v, oi_v):
    wid = lax.axis_index("s") * 2 + lax.axis_index("c")
    iota = _iota16()
    ones = jnp.ones((16,), jnp.int32)

    def per_row(ri, _):
        row = wid * RPW + ri
        pltpu.sync_copy(scores_hbm.at[row], row_v)

        _zero_hist(hist_v)

        def h1(i, _):
            u = _ukey(row_v[pl.ds(i * 16, 16)])
            b = (u >> jnp.uint32(24)).astype(jnp.int32)
            plsc.addupdate_scatter(hist_v, [iota * 256 + b], ones)
            return 0

        lax.fori_loop(0, NV, h1, 0)
        _totals(hist_v, tot_v)
        b1, n_gt = _find_bucket(tot_v, K, iota)

        def comp(i, carry):
            off_hi, off_mid = carry
            u = _ukey(row_v[pl.ds(i * 16, 16)])
            b = (u >> jnp.uint32(24)).astype(jnp.int32)
            gidx = i * 16 + iota
            m_hi = b > b1
            m_mid = b == b1
            plsc.store_compressed(fin_v.at[pl.ds(off_hi, 16)], gidx, mask=m_hi)
            plsc.store_compressed(mid_v.at[pl.ds(off_mid, 16)], gidx, mask=m_mid)
            return off_hi + _pcount(m_hi), off_mid + _pcount(m_mid)

        n_hi, n_mid = lax.fori_loop(0, NV, comp, (0, 0))
        nv_mid = (n_mid + 15) // 16

        prefix = b1.astype(jnp.uint32)
        for shift in (16, 8, 0):
            _zero_hist(hist_v)
            needed = K - n_gt

            def rh(i, _, shift=shift, prefix=prefix):
                idx = mid_v[pl.ds(i * 16, 16)]
                valid = (i * 16 + iota) < n_mid
                vals = plsc.load_gather(row_v, [idx], mask=valid)
                u = _ukey(vals)
                match = jnp.logical_and(
                    valid, (u >> jnp.uint32(shift + 8)) == prefix)
                byte = ((u >> jnp.uint32(shift)) & jnp.uint32(0xFF)).astype(
                    jnp.int32)
                plsc.addupdate_scatter(hist_v, [iota * 256 + byte], ones,
                                       mask=match)
                return 0

            lax.fori_loop(0, nv_mid, rh, 0)
            _totals(hist_v, tot_v)
            bs, add_gt = _find_bucket(tot_v, needed, iota)
            n_gt = n_gt + add_gt
            prefix = (prefix << jnp.uint32(8)) | bs.astype(jnp.uint32)
        t_key = prefix

        def gt_scan(i, off):
            idx = mid_v[pl.ds(i * 16, 16)]
            valid = (i * 16 + iota) < n_mid
            u = _ukey(plsc.load_gather(row_v, [idx], mask=valid))
            m = jnp.logical_and(valid, u > t_key)
            plsc.store_compressed(fin_v.at[pl.ds(off, 16)], idx, mask=m)
            return off + _pcount(m)

        off = lax.fori_loop(0, nv_mid, gt_scan, n_hi)

        def eq_scan(i, off):
            idx = mid_v[pl.ds(i * 16, 16)]
            valid = (i * 16 + iota) < n_mid
            u = _ukey(plsc.load_gather(row_v, [idx], mask=valid))
            m = jnp.logical_and(valid, u == t_key)
            pc = _pcount(m)

            @pl.when(off < K)
            def _():
                plsc.store_compressed(fin_v.at[pl.ds(off, 16)], idx, mask=m)

            return jnp.where(off < K, off + pc, off)

        lax.fori_loop(0, nv_mid, eq_scan, off)

        ks, js = [], []
        for r in range(16):
            idxv = fin_v[pl.ds(r * 16, 16)]
            ks.append(_ukey(plsc.load_gather(row_v, [idxv])))
            js.append(idxv)
        ks, js = _bitonic256(ks, js, iota)

        for r in range(16):
            ov_v[pl.ds(r * 16, 16)] = plsc.load_gather(row_v, [js[r]])
            oi_v[pl.ds(r * 16, 16)] = js[r]
        pltpu.sync_copy(ov_v, vals_hbm.at[row])
        pltpu.sync_copy(oi_v, idx_hbm.at[row])
        return 0

    lax.fori_loop(0, RPW, per_row, 0)


@jax.jit
def kernel(scores):
    mesh = plsc.VectorSubcoreMesh(core_axis_name="c", subcore_axis_name="s")
    f = functools.partial(
        pl.kernel,
        mesh=mesh,
        out_type=(
            jax.ShapeDtypeStruct((ROWS, K), jnp.float32),
            jax.ShapeDtypeStruct((ROWS, K), jnp.int32),
        ),
        compiler_params=pltpu.CompilerParams(needs_layout_passes=False),
        scratch_types=[
            pltpu.VMEM((N,), jnp.float32),
            pltpu.VMEM((N + 16,), jnp.int32),
            pltpu.VMEM((4096,), jnp.int32),
            pltpu.VMEM((256,), jnp.int32),
            pltpu.VMEM((288,), jnp.int32),
            pltpu.VMEM((K,), jnp.float32),
            pltpu.VMEM((K,), jnp.int32),
        ],
    )(_body)
    return f(scores)


# speedup vs baseline: 1.0149x; 1.0149x over previous
import functools

import jax
import jax.numpy as jnp
from jax import lax
from jax.experimental import pallas as pl
from jax.experimental.pallas import tpu as pltpu
from jax.experimental.pallas import tpu_sc as plsc

K = 256
N = 32768
NV = N // 16
ROWS = 64
NWORK = 32
RPW = ROWS // NWORK


def _iota16():
    return lax.broadcasted_iota(jnp.int32, (16,), 0)


def _ukey(x):
    b = lax.bitcast_convert_type(x, jnp.uint32)
    neg = b >> jnp.uint32(31)
    flip = jnp.where(neg == jnp.uint32(1), jnp.uint32(0xFFFFFFFF),
                     jnp.uint32(0x80000000))
    return b ^ flip


def _pcount(m):
    return jnp.max(plsc.all_reduce_population_count(m))


def _find_bucket(tot_ref, needed, iota):
    def body(gi, carry):
        found, b_star, n_gt, above = carry
        g = 15 - gi
        t = tot_ref[pl.ds(g * 16, 16)]
        r = lax.rev(t, (0,))
        c = plsc.cumsum(r)
        s_ge = c + above
        l_star = _pcount(s_ge < needed)
        in_group = l_star < 16
        bs_g = g * 16 + 15 - l_star
        cprev = jnp.sum(jnp.where(iota == (l_star - 1), c, 0))
        s_gt = above + cprev
        take = jnp.logical_and(found == 0, in_group)
        return (jnp.where(take, 1, found),
                jnp.where(take, bs_g, b_star),
                jnp.where(take, s_gt, n_gt),
                above + jnp.sum(t))

    _, b_star, n_gt, _ = lax.fori_loop(0, 16, body, (0, 0, 0, 0))
    return b_star, n_gt


def _totals(hist_ref, tot_ref):
    def body(g, _):
        acc = hist_ref[pl.ds(g * 16, 16)]
        for lane in range(1, 16):
            acc = acc + hist_ref[pl.ds(lane * 256 + g * 16, 16)]
        tot_ref[pl.ds(g * 16, 16)] = acc
        return 0

    lax.fori_loop(0, 16, body, 0)


def _zero_hist(hist_ref):
    z = jnp.zeros((16,), jnp.int32)

    def body(i, _):
        hist_ref[pl.ds(i * 16, 16)] = z
        return 0

    lax.fori_loop(0, 256, body, 0, unroll=8)


def _perm16(x, perm):
    dnums = lax.GatherDimensionNumbers(
        offset_dims=(), collapsed_slice_dims=(0,), start_index_map=(0,))
    return lax.gather(x, perm.reshape(16, 1), dnums, (1,),
                      mode=lax.GatherScatterMode.PROMISE_IN_BOUNDS,
                      unique_indices=True)


def _cmp_gt(ka, ia, kb, ib):
    return (ka > kb) | ((ka == kb) & (ia < ib))


def _bitonic256(ks, js, iota):
    for kk in (2, 4, 8, 16, 32, 64, 128, 256):
        j = kk // 2
        while j >= 1:
            if j >= 16:
                jj = j // 16
                for r in range(16):
                    if r & jj:
                        continue
                    a, b = r, r | jj
                    desc = ((r * 16) & kk) == 0
                    g = _cmp_gt(ks[a], js[a], ks[b], js[b])
                    if desc:
                        sel = g
                    else:
                        sel = jnp.logical_not(g)
                    nka = jnp.where(sel, ks[a], ks[b])
                    nja = jnp.where(sel, js[a], js[b])
                    nkb = jnp.where(sel, ks[b], ks[a])
                    njb = jnp.where(sel, js[b], js[a])
                    ks[a], js[a], ks[b], js[b] = nka, nja, nkb, njb
            else:
                perm = iota ^ j
                is_lo = (iota & j) == 0
                for r in range(16):
                    if kk >= 32 or kk == 16:
                        desc_static = ((r * 16) & kk) == 0
                        want_greater = is_lo if desc_static else (~is_lo)
                    else:
                        desc = ((r * 16 + iota) & kk) == 0
                        want_greater = jnp.logical_not(jnp.logical_xor(desc, is_lo))
                    kp = _perm16(ks[r], perm)
                    ip = _perm16(js[r], perm)
                    g = _cmp_gt(ks[r], js[r], kp, ip)
                    choose_self = want_greater == g
                    ks[r] = jnp.where(choose_self, ks[r], kp)
                    js[r] = jnp.where(choose_self, js[r], ip)
            j //= 2
    return ks, js


def _body(scores_hbm, vals_hbm, idx_hbm, row_v, mid_v, hist_v, tot_v, fin_v,
          ov_v, oi_v):
    wid = lax.axis_index("s") * 2 + lax.axis_index("c")
    iota = _iota16()
    ones = jnp.ones((16,), jnp.int32)

    def per_row(ri, _):
        row = wid * RPW + ri
        pltpu.sync_copy(scores_hbm.at[row], row_v)

        _zero_hist(hist_v)

        def h1(i, _):
            u = _ukey(row_v[pl.ds(i * 16, 16)])
            b = (u >> jnp.uint32(24)).astype(jnp.int32)
            plsc.addupdate_scatter(hist_v, [iota * 256 + b], ones)
            return 0

        lax.fori_loop(0, NV, h1, 0, unroll=8)
        _totals(hist_v, tot_v)
        b1, n_gt = _find_bucket(tot_v, K, iota)

        def comp(i, carry):
            off_hi, off_mid = carry
            u = _ukey(row_v[pl.ds(i * 16, 16)])
            b = (u >> jnp.uint32(24)).astype(jnp.int32)
            gidx = i * 16 + iota
            m_hi = b > b1
            m_mid = b == b1
            plsc.store_compressed(fin_v.at[pl.ds(off_hi, 16)], gidx, mask=m_hi)
            plsc.store_compressed(mid_v.at[pl.ds(off_mid, 16)], gidx, mask=m_mid)
            return off_hi + _pcount(m_hi), off_mid + _pcount(m_mid)

        n_hi, n_mid = lax.fori_loop(0, NV, comp, (0, 0))
        nv_mid = (n_mid + 15) // 16

        prefix = b1.astype(jnp.uint32)
        z16 = jnp.zeros((16,), jnp.int32)
        for shift in (20, 16, 12, 8, 4, 0):
            for lane in range(16):
                hist_v[pl.ds(lane * 16, 16)] = z16
            needed = K - n_gt

            def rh(i, _, shift=shift, prefix=prefix):
                idx = mid_v[pl.ds(i * 16, 16)]
                valid = (i * 16 + iota) < n_mid
                vals = plsc.load_gather(row_v, [idx], mask=valid)
                u = _ukey(vals)
                match = jnp.logical_and(
                    valid, (u >> jnp.uint32(shift + 4)) == prefix)
                dig = ((u >> jnp.uint32(shift)) & jnp.uint32(0xF)).astype(
                    jnp.int32)
                plsc.addupdate_scatter(hist_v, [iota * 16 + dig], ones,
                                       mask=match)
                return 0

            lax.fori_loop(0, nv_mid, rh, 0)
            acc = hist_v[pl.ds(0, 16)]
            for lane in range(1, 16):
                acc = acc + hist_v[pl.ds(lane * 16, 16)]
            c = plsc.cumsum(lax.rev(acc, (0,)))
            l_star = _pcount(c < needed)
            bs = 15 - l_star
            cprev = jnp.sum(jnp.where(iota == (l_star - 1), c, 0))
            n_gt = n_gt + cprev
            prefix = (prefix << jnp.uint32(4)) | bs.astype(jnp.uint32)
        t_key = prefix

        def gt_scan(i, off):
            idx = mid_v[pl.ds(i * 16, 16)]
            valid = (i * 16 + iota) < n_mid
            u = _ukey(plsc.load_gather(row_v, [idx], mask=valid))
            m = jnp.logical_and(valid, u > t_key)
            plsc.store_compressed(fin_v.at[pl.ds(off, 16)], idx, mask=m)
            return off + _pcount(m)

        off = lax.fori_loop(0, nv_mid, gt_scan, n_hi)

        def eq_scan(i, off):
            idx = mid_v[pl.ds(i * 16, 16)]
            valid = (i * 16 + iota) < n_mid
            u = _ukey(plsc.load_gather(row_v, [idx], mask=valid))
            m = jnp.logical_and(valid, u == t_key)
            pc = _pcount(m)

            @pl.when(off < K)
            def _():
                plsc.store_compressed(fin_v.at[pl.ds(off, 16)], idx, mask=m)

            return jnp.where(off < K, off + pc, off)

        lax.fori_loop(0, nv_mid, eq_scan, off)

        ks, js = [], []
        for r in range(16):
            idxv = fin_v[pl.ds(r * 16, 16)]
            ks.append(_ukey(plsc.load_gather(row_v, [idxv])))
            js.append(idxv)
        ks, js = _bitonic256(ks, js, iota)

        for r in range(16):
            ov_v[pl.ds(r * 16, 16)] = plsc.load_gather(row_v, [js[r]])
            oi_v[pl.ds(r * 16, 16)] = js[r]
        pltpu.sync_copy(ov_v, vals_hbm.at[row])
        pltpu.sync_copy(oi_v, idx_hbm.at[row])
        return 0

    lax.fori_loop(0, RPW, per_row, 0)


@jax.jit
def kernel(scores):
    mesh = plsc.VectorSubcoreMesh(core_axis_name="c", subcore_axis_name="s")
    f = functools.partial(
        pl.kernel,
        mesh=mesh,
        out_type=(
            jax.ShapeDtypeStruct((ROWS, K), jnp.float32),
            jax.ShapeDtypeStruct((ROWS, K), jnp.int32),
        ),
        compiler_params=pltpu.CompilerParams(needs_layout_passes=False),
        scratch_types=[
            pltpu.VMEM((N,), jnp.float32),
            pltpu.VMEM((N + 16,), jnp.int32),
            pltpu.VMEM((4096,), jnp.int32),
            pltpu.VMEM((256,), jnp.int32),
            pltpu.VMEM((288,), jnp.int32),
            pltpu.VMEM((K,), jnp.float32),
            pltpu.VMEM((K,), jnp.int32),
        ],
    )(_body)
    return f(scores)


# speedup vs baseline: 1.0381x; 1.0228x over previous
import functools

import jax
import jax.numpy as jnp
from jax import lax
from jax.experimental import pallas as pl
from jax.experimental.pallas import tpu as pltpu
from jax.experimental.pallas import tpu_sc as plsc

K = 256
N = 32768
NV = N // 16
ROWS = 64
NWORK = 32
RPW = ROWS // NWORK


def _iota16():
    return lax.broadcasted_iota(jnp.int32, (16,), 0)


def _ukey(x):
    b = lax.bitcast_convert_type(x, jnp.uint32)
    neg = b >> jnp.uint32(31)
    flip = jnp.where(neg == jnp.uint32(1), jnp.uint32(0xFFFFFFFF),
                     jnp.uint32(0x80000000))
    return b ^ flip


def _pcount(m):
    return plsc.all_reduce_population_count(m)[0]


def _find_bucket(tot_ref, needed, iota):
    def body(gi, carry):
        found, b_star, n_gt, above = carry
        g = 15 - gi
        t = tot_ref[pl.ds(g * 16, 16)]
        r = lax.rev(t, (0,))
        c = plsc.cumsum(r)
        s_ge = c + above
        l_star = _pcount(s_ge < needed)
        in_group = l_star < 16
        bs_g = g * 16 + 15 - l_star
        cprev = jnp.sum(jnp.where(iota == (l_star - 1), c, 0))
        s_gt = above + cprev
        take = jnp.logical_and(found == 0, in_group)
        return (jnp.where(take, 1, found),
                jnp.where(take, bs_g, b_star),
                jnp.where(take, s_gt, n_gt),
                above + jnp.sum(t))

    _, b_star, n_gt, _ = lax.fori_loop(0, 16, body, (0, 0, 0, 0))
    return b_star, n_gt


NH = 4


def _totals(hist_ref, tot_ref):
    def body(g, _):
        acc = hist_ref[pl.ds(g * 16, 16)]
        for c in range(NH):
            for lane in range(16):
                if c == 0 and lane == 0:
                    continue
                acc = acc + hist_ref[pl.ds(c * 4096 + lane * 256 + g * 16, 16)]
        tot_ref[pl.ds(g * 16, 16)] = acc
        return 0

    lax.fori_loop(0, 16, body, 0)


def _zero_hist(hist_ref):
    z = jnp.zeros((16,), jnp.int32)

    def body(i, _):
        hist_ref[pl.ds(i * 16, 16)] = z
        return 0

    lax.fori_loop(0, NH * 256, body, 0, unroll=8)


def _perm16(x, perm):
    dnums = lax.GatherDimensionNumbers(
        offset_dims=(), collapsed_slice_dims=(0,), start_index_map=(0,))
    return lax.gather(x, perm.reshape(16, 1), dnums, (1,),
                      mode=lax.GatherScatterMode.PROMISE_IN_BOUNDS,
                      unique_indices=True)


def _cmp_gt(ka, ia, kb, ib):
    return (ka > kb) | ((ka == kb) & (ia < ib))


def _bitonic256(ks, js, iota):
    for kk in (2, 4, 8, 16, 32, 64, 128, 256):
        j = kk // 2
        while j >= 1:
            if j >= 16:
                jj = j // 16
                for r in range(16):
                    if r & jj:
                        continue
                    a, b = r, r | jj
                    desc = ((r * 16) & kk) == 0
                    g = _cmp_gt(ks[a], js[a], ks[b], js[b])
                    if desc:
                        sel = g
                    else:
                        sel = jnp.logical_not(g)
                    nka = jnp.where(sel, ks[a], ks[b])
                    nja = jnp.where(sel, js[a], js[b])
                    nkb = jnp.where(sel, ks[b], ks[a])
                    njb = jnp.where(sel, js[b], js[a])
                    ks[a], js[a], ks[b], js[b] = nka, nja, nkb, njb
            else:
                perm = iota ^ j
                is_lo = (iota & j) == 0
                for r in range(16):
                    if kk >= 32 or kk == 16:
                        desc_static = ((r * 16) & kk) == 0
                        want_greater = is_lo if desc_static else (~is_lo)
                    else:
                        desc = ((r * 16 + iota) & kk) == 0
                        want_greater = jnp.logical_not(jnp.logical_xor(desc, is_lo))
                    kp = _perm16(ks[r], perm)
                    ip = _perm16(js[r], perm)
                    g = _cmp_gt(ks[r], js[r], kp, ip)
                    choose_self = want_greater == g
                    ks[r] = jnp.where(choose_self, ks[r], kp)
                    js[r] = jnp.where(choose_self, js[r], ip)
            j //= 2
    return ks, js


def _body(scores_hbm, vals_hbm, idx_hbm, row_v, mid_v, hist_v, tot_v, fin_v,
          ov_v, oi_v):
    wid = lax.axis_index("s") * 2 + lax.axis_index("c")
    iota = _iota16()
    ones = jnp.ones((16,), jnp.int32)

    def per_row(ri, _):
        row = wid * RPW + ri
        pltpu.sync_copy(scores_hbm.at[row], row_v)

        _zero_hist(hist_v)

        def h1(i, _):
            for c in range(NH):
                u = _ukey(row_v[pl.ds((i * NH + c) * 16, 16)])
                b = (u >> jnp.uint32(24)).astype(jnp.int32)
                plsc.addupdate_scatter(
                    hist_v, [c * 4096 + iota * 256 + b], ones)
            return 0

        lax.fori_loop(0, NV // NH, h1, 0, unroll=2)
        _totals(hist_v, tot_v)
        b1, n_gt = _find_bucket(tot_v, K, iota)

        def comp(i, carry):
            off_hi, off_mid = carry
            u = _ukey(row_v[pl.ds(i * 16, 16)])
            b = (u >> jnp.uint32(24)).astype(jnp.int32)
            gidx = i * 16 + iota
            m_hi = b > b1
            m_mid = b == b1
            plsc.store_compressed(fin_v.at[pl.ds(off_hi, 16)], gidx, mask=m_hi)
            plsc.store_compressed(mid_v.at[pl.ds(off_mid, 16)], gidx, mask=m_mid)
            return off_hi + _pcount(m_hi), off_mid + _pcount(m_mid)

        n_hi, n_mid = lax.fori_loop(0, NV, comp, (0, 0))
        nv_mid = (n_mid + 15) // 16

        prefix = b1.astype(jnp.uint32)
        z16 = jnp.zeros((16,), jnp.int32)
        for shift in (20, 16, 12, 8, 4, 0):
            for lane in range(16):
                hist_v[pl.ds(lane * 16, 16)] = z16
            needed = K - n_gt

            def rh(i, _, shift=shift, prefix=prefix):
                idx = mid_v[pl.ds(i * 16, 16)]
                valid = (i * 16 + iota) < n_mid
                vals = plsc.load_gather(row_v, [idx], mask=valid)
                u = _ukey(vals)
                match = jnp.logical_and(
                    valid, (u >> jnp.uint32(shift + 4)) == prefix)
                dig = ((u >> jnp.uint32(shift)) & jnp.uint32(0xF)).astype(
                    jnp.int32)
                plsc.addupdate_scatter(hist_v, [iota * 16 + dig], ones,
                                       mask=match)
                return 0

            lax.fori_loop(0, nv_mid, rh, 0)
            acc = hist_v[pl.ds(0, 16)]
            for lane in range(1, 16):
                acc = acc + hist_v[pl.ds(lane * 16, 16)]
            c = plsc.cumsum(lax.rev(acc, (0,)))
            l_star = _pcount(c < needed)
            bs = 15 - l_star
            cprev = jnp.sum(jnp.where(iota == (l_star - 1), c, 0))
            n_gt = n_gt + cprev
            prefix = (prefix << jnp.uint32(4)) | bs.astype(jnp.uint32)
        t_key = prefix

        def gt_scan(i, off):
            idx = mid_v[pl.ds(i * 16, 16)]
            valid = (i * 16 + iota) < n_mid
            u = _ukey(plsc.load_gather(row_v, [idx], mask=valid))
            m = jnp.logical_and(valid, u > t_key)
            plsc.store_compressed(fin_v.at[pl.ds(off, 16)], idx, mask=m)
            return off + _pcount(m)

        off = lax.fori_loop(0, nv_mid, gt_scan, n_hi)

        def eq_scan(i, off):
            idx = mid_v[pl.ds(i * 16, 16)]
            valid = (i * 16 + iota) < n_mid
            u = _ukey(plsc.load_gather(row_v, [idx], mask=valid))
            m = jnp.logical_and(valid, u == t_key)
            pc = _pcount(m)

            @pl.when(off < K)
            def _():
                plsc.store_compressed(fin_v.at[pl.ds(off, 16)], idx, mask=m)

            return jnp.where(off < K, off + pc, off)

        lax.fori_loop(0, nv_mid, eq_scan, off)

        ks, js = [], []
        for r in range(16):
            idxv = fin_v[pl.ds(r * 16, 16)]
            ks.append(_ukey(plsc.load_gather(row_v, [idxv])))
            js.append(idxv)
        ks, js = _bitonic256(ks, js, iota)

        for r in range(16):
            ov_v[pl.ds(r * 16, 16)] = plsc.load_gather(row_v, [js[r]])
            oi_v[pl.ds(r * 16, 16)] = js[r]
        pltpu.sync_copy(ov_v, vals_hbm.at[row])
        pltpu.sync_copy(oi_v, idx_hbm.at[row])
        return 0

    lax.fori_loop(0, RPW, per_row, 0)


@jax.jit
def kernel(scores):
    mesh = plsc.VectorSubcoreMesh(core_axis_name="c", subcore_axis_name="s")
    f = functools.partial(
        pl.kernel,
        mesh=mesh,
        out_type=(
            jax.ShapeDtypeStruct((ROWS, K), jnp.float32),
            jax.ShapeDtypeStruct((ROWS, K), jnp.int32),
        ),
        compiler_params=pltpu.CompilerParams(needs_layout_passes=False),
        scratch_types=[
            pltpu.VMEM((N,), jnp.float32),
            pltpu.VMEM((N + 16,), jnp.int32),
            pltpu.VMEM((NH * 4096,), jnp.int32),
            pltpu.VMEM((256,), jnp.int32),
            pltpu.VMEM((288,), jnp.int32),
            pltpu.VMEM((K,), jnp.float32),
            pltpu.VMEM((K,), jnp.int32),
        ],
    )(_body)
    return f(scores)


# speedup vs baseline: 1.4932x; 1.4384x over previous
import functools

import jax
import jax.numpy as jnp
from jax import lax
from jax.experimental import pallas as pl
from jax.experimental.pallas import tpu as pltpu
from jax.experimental.pallas import tpu_sc as plsc

K = 256
N = 32768
NV = N // 16
PRE = 8
NVM = NV // PRE
ROWS = 64
NWORK = 32
RPW = ROWS // NWORK


def _iota16():
    return lax.broadcasted_iota(jnp.int32, (16,), 0)


def _ukey(x):
    b = lax.bitcast_convert_type(x, jnp.uint32)
    neg = b >> jnp.uint32(31)
    flip = jnp.where(neg == jnp.uint32(1), jnp.uint32(0xFFFFFFFF),
                     jnp.uint32(0x80000000))
    return b ^ flip


def _pcount(m):
    return plsc.all_reduce_population_count(m)[0]


def _find_bucket(tot_ref, needed, iota):
    def body(gi, carry):
        found, b_star, above = carry
        g = 15 - gi
        t = tot_ref[pl.ds(g * 16, 16)]
        c = plsc.cumsum(lax.rev(t, (0,)))
        s_ge = c + above
        l_star = _pcount(s_ge < needed)
        in_group = l_star < 16
        bs_g = g * 16 + 15 - l_star
        take = jnp.logical_and(found == 0, in_group)
        return (jnp.where(take, 1, found),
                jnp.where(take, bs_g, b_star),
                above + jnp.sum(t))

    _, b_star, _ = lax.fori_loop(0, 16, body, (0, 0, 0))
    return b_star


def _totals(hist_ref, tot_ref):
    def body(g, _):
        acc = hist_ref[pl.ds(g * 16, 16)]
        for lane in range(1, 16):
            acc = acc + hist_ref[pl.ds(lane * 256 + g * 16, 16)]
        tot_ref[pl.ds(g * 16, 16)] = acc
        return 0

    lax.fori_loop(0, 16, body, 0)


def _zero_hist(hist_ref):
    z = jnp.zeros((16,), jnp.int32)

    def body(i, _):
        hist_ref[pl.ds(i * 16, 16)] = z
        return 0

    lax.fori_loop(0, 256, body, 0, unroll=8)


def _perm16(x, perm):
    dnums = lax.GatherDimensionNumbers(
        offset_dims=(), collapsed_slice_dims=(0,), start_index_map=(0,))
    return lax.gather(x, perm.reshape(16, 1), dnums, (1,),
                      mode=lax.GatherScatterMode.PROMISE_IN_BOUNDS,
                      unique_indices=True)


def _cmp_gt(ka, ia, kb, ib):
    return (ka > kb) | ((ka == kb) & (ia < ib))


def _bitonic256(ks, js, iota):
    for kk in (2, 4, 8, 16, 32, 64, 128, 256):
        j = kk // 2
        while j >= 1:
            if j >= 16:
                jj = j // 16
                for r in range(16):
                    if r & jj:
                        continue
                    a, b = r, r | jj
                    desc = ((r * 16) & kk) == 0
                    g = _cmp_gt(ks[a], js[a], ks[b], js[b])
                    if desc:
                        sel = g
                    else:
                        sel = jnp.logical_not(g)
                    nka = jnp.where(sel, ks[a], ks[b])
                    nja = jnp.where(sel, js[a], js[b])
                    nkb = jnp.where(sel, ks[b], ks[a])
                    njb = jnp.where(sel, js[b], js[a])
                    ks[a], js[a], ks[b], js[b] = nka, nja, nkb, njb
            else:
                perm = iota ^ j
                is_lo = (iota & j) == 0
                for r in range(16):
                    if kk >= 32 or kk == 16:
                        desc_static = ((r * 16) & kk) == 0
                        want_greater = is_lo if desc_static else (~is_lo)
                    else:
                        desc = ((r * 16 + iota) & kk) == 0
                        want_greater = jnp.logical_not(jnp.logical_xor(desc, is_lo))
                    kp = _perm16(ks[r], perm)
                    ip = _perm16(js[r], perm)
                    g = _cmp_gt(ks[r], js[r], kp, ip)
                    choose_self = want_greater == g
                    ks[r] = jnp.where(choose_self, ks[r], kp)
                    js[r] = jnp.where(choose_self, js[r], ip)
            j //= 2
    return ks, js


def _body(scores_hbm, vals_hbm, idx_hbm, row_v, mid_v, mval_v, mx_v, hist_v,
          tot_v, fin_v, ov_v, oi_v):
    wid = lax.axis_index("s") * 2 + lax.axis_index("c")
    iota = _iota16()
    ones = jnp.ones((16,), jnp.int32)

    def per_row(ri, _):
        row = wid * RPW + ri
        pltpu.sync_copy(scores_hbm.at[row], row_v)

        def pre(i, _):
            m = row_v[pl.ds(i * PRE * 16, 16)]
            for c in range(1, PRE):
                m = jnp.maximum(m, row_v[pl.ds((i * PRE + c) * 16, 16)])
            mx_v[pl.ds(i * 16, 16)] = m
            return 0

        lax.fori_loop(0, NVM, pre, 0, unroll=4)

        _zero_hist(hist_v)

        def h1(i, _):
            u = _ukey(mx_v[pl.ds(i * 16, 16)])
            b = (u >> jnp.uint32(24)).astype(jnp.int32)
            plsc.addupdate_scatter(hist_v, [iota * 256 + b], ones)
            return 0

        lax.fori_loop(0, NVM, h1, 0, unroll=4)
        _totals(hist_v, tot_v)
        b1 = _find_bucket(tot_v, K, iota)
        t_scr = b1.astype(jnp.uint32) << jnp.uint32(24)

        def comp(i, off):
            u = _ukey(row_v[pl.ds(i * 16, 16)])
            m = u >= t_scr
            plsc.store_compressed(mid_v.at[pl.ds(off, 16)],
                                  i * 16 + iota, mask=m)
            return off + _pcount(m)

        n_mid = lax.fori_loop(0, NV, comp, 0)
        nv_mid = (n_mid + 15) // 16

        def cache(i, _):
            idx = mid_v[pl.ds(i * 16, 16)]
            valid = (i * 16 + iota) < n_mid
            mval_v[pl.ds(i * 16, 16)] = plsc.load_gather(
                row_v, [idx], mask=valid)
            return 0

        lax.fori_loop(0, nv_mid, cache, 0)

        n_gt = 0
        prefix = jnp.uint32(0)
        z16 = jnp.zeros((16,), jnp.int32)
        for shift in (28, 24, 20, 16, 12, 8, 4, 0):
            for lane in range(16):
                hist_v[pl.ds(lane * 16, 16)] = z16
            needed = K - n_gt

            def rh(i, _, shift=shift, prefix=prefix):
                u = _ukey(mval_v[pl.ds(i * 16, 16)])
                valid = (i * 16 + iota) < n_mid
                if shift == 28:
                    match = valid
                else:
                    match = jnp.logical_and(
                        valid, (u >> jnp.uint32(shift + 4)) == prefix)
                dig = ((u >> jnp.uint32(shift)) & jnp.uint32(0xF)).astype(
                    jnp.int32)
                plsc.addupdate_scatter(hist_v, [iota * 16 + dig], ones,
                                       mask=match)
                return 0

            lax.fori_loop(0, nv_mid, rh, 0)
            acc = hist_v[pl.ds(0, 16)]
            for lane in range(1, 16):
                acc = acc + hist_v[pl.ds(lane * 16, 16)]
            c = plsc.cumsum(lax.rev(acc, (0,)))
            l_star = _pcount(c < needed)
            bs = 15 - l_star
            cprev = jnp.sum(jnp.where(iota == (l_star - 1), c, 0))
            n_gt = n_gt + cprev
            prefix = (prefix << jnp.uint32(4)) | bs.astype(jnp.uint32)
        t_key = prefix

        def gt_scan(i, off):
            u = _ukey(mval_v[pl.ds(i * 16, 16)])
            valid = (i * 16 + iota) < n_mid
            m = jnp.logical_and(valid, u > t_key)
            plsc.store_compressed(fin_v.at[pl.ds(off, 16)],
                                  mid_v[pl.ds(i * 16, 16)], mask=m)
            return off + _pcount(m)

        off = lax.fori_loop(0, nv_mid, gt_scan, 0)

        def eq_scan(i, off):
            u = _ukey(mval_v[pl.ds(i * 16, 16)])
            valid = (i * 16 + iota) < n_mid
            m = jnp.logical_and(valid, u == t_key)
            pc = _pcount(m)

            @pl.when(off < K)
            def _():
                plsc.store_compressed(fin_v.at[pl.ds(off, 16)],
                                      mid_v[pl.ds(i * 16, 16)], mask=m)

            return jnp.where(off < K, off + pc, off)

        lax.fori_loop(0, nv_mid, eq_scan, off)

        ks, js = [], []
        for r in range(16):
            idxv = fin_v[pl.ds(r * 16, 16)]
            ks.append(_ukey(plsc.load_gather(row_v, [idxv])))
            js.append(idxv)
        ks, js = _bitonic256(ks, js, iota)

        for r in range(16):
            ov_v[pl.ds(r * 16, 16)] = plsc.load_gather(row_v, [js[r]])
            oi_v[pl.ds(r * 16, 16)] = js[r]
        pltpu.sync_copy(ov_v, vals_hbm.at[row])
        pltpu.sync_copy(oi_v, idx_hbm.at[row])
        return 0

    lax.fori_loop(0, RPW, per_row, 0)


@jax.jit
def kernel(scores):
    mesh = plsc.VectorSubcoreMesh(core_axis_name="c", subcore_axis_name="s")
    f = functools.partial(
        pl.kernel,
        mesh=mesh,
        out_type=(
            jax.ShapeDtypeStruct((ROWS, K), jnp.float32),
            jax.ShapeDtypeStruct((ROWS, K), jnp.int32),
        ),
        compiler_params=pltpu.CompilerParams(
            needs_layout_passes=False, disable_bounds_checks=True),
        scratch_types=[
            pltpu.VMEM((N,), jnp.float32),
            pltpu.VMEM((N + 16,), jnp.int32),
            pltpu.VMEM((N + 16,), jnp.float32),
            pltpu.VMEM((N // PRE,), jnp.float32),
            pltpu.VMEM((4096,), jnp.int32),
            pltpu.VMEM((256,), jnp.int32),
            pltpu.VMEM((288,), jnp.int32),
            pltpu.VMEM((K,), jnp.float32),
            pltpu.VMEM((K,), jnp.int32),
        ],
    )(_body)
    return f(scores)


# speedup vs baseline: 1.5230x; 1.0200x over previous
import functools

import jax
import jax.numpy as jnp
from jax import lax
from jax.experimental import pallas as pl
from jax.experimental.pallas import tpu as pltpu
from jax.experimental.pallas import tpu_sc as plsc

K = 256
N = 32768
NV = N // 16
PRE = 8
NVM = NV // PRE
ROWS = 64
NWORK = 32
RPW = ROWS // NWORK
QC = 4
QN = NV // QC
QCAP = QN * 16 + 16


def _iota16():
    return lax.broadcasted_iota(jnp.int32, (16,), 0)


def _ukey(x):
    b = lax.bitcast_convert_type(x, jnp.uint32)
    neg = b >> jnp.uint32(31)
    flip = jnp.where(neg == jnp.uint32(1), jnp.uint32(0xFFFFFFFF),
                     jnp.uint32(0x80000000))
    return b ^ flip


def _pcount(m):
    return plsc.all_reduce_population_count(m)[0]


def _find_bucket(tot_ref, needed, iota):
    def body(gi, carry):
        found, b_star, above = carry
        g = 15 - gi
        t = tot_ref[pl.ds(g * 16, 16)]
        c = plsc.cumsum(lax.rev(t, (0,)))
        s_ge = c + above
        l_star = _pcount(s_ge < needed)
        in_group = l_star < 16
        bs_g = g * 16 + 15 - l_star
        take = jnp.logical_and(found == 0, in_group)
        return (jnp.where(take, 1, found),
                jnp.where(take, bs_g, b_star),
                above + jnp.sum(t))

    _, b_star, _ = lax.fori_loop(0, 16, body, (0, 0, 0))
    return b_star


def _totals(hist_ref, tot_ref):
    def body(g, _):
        acc = hist_ref[pl.ds(g * 16, 16)]
        for lane in range(1, 16):
            acc = acc + hist_ref[pl.ds(lane * 256 + g * 16, 16)]
        tot_ref[pl.ds(g * 16, 16)] = acc
        return 0

    lax.fori_loop(0, 16, body, 0)


def _zero_hist(hist_ref):
    z = jnp.zeros((16,), jnp.int32)

    def body(i, _):
        hist_ref[pl.ds(i * 16, 16)] = z
        return 0

    lax.fori_loop(0, 256, body, 0, unroll=8)


def _perm16(x, perm):
    dnums = lax.GatherDimensionNumbers(
        offset_dims=(), collapsed_slice_dims=(0,), start_index_map=(0,))
    return lax.gather(x, perm.reshape(16, 1), dnums, (1,),
                      mode=lax.GatherScatterMode.PROMISE_IN_BOUNDS,
                      unique_indices=True)


def _cmp_gt(ka, ia, kb, ib):
    return (ka > kb) | ((ka == kb) & (ia < ib))


def _bitonic256(ks, js, iota):
    for kk in (2, 4, 8, 16, 32, 64, 128, 256):
        j = kk // 2
        while j >= 1:
            if j >= 16:
                jj = j // 16
                for r in range(16):
                    if r & jj:
                        continue
                    a, b = r, r | jj
                    desc = ((r * 16) & kk) == 0
                    g = _cmp_gt(ks[a], js[a], ks[b], js[b])
                    if desc:
                        sel = g
                    else:
                        sel = jnp.logical_not(g)
                    nka = jnp.where(sel, ks[a], ks[b])
                    nja = jnp.where(sel, js[a], js[b])
                    nkb = jnp.where(sel, ks[b], ks[a])
                    njb = jnp.where(sel, js[b], js[a])
                    ks[a], js[a], ks[b], js[b] = nka, nja, nkb, njb
            else:
                perm = iota ^ j
                is_lo = (iota & j) == 0
                for r in range(16):
                    if kk >= 32 or kk == 16:
                        desc_static = ((r * 16) & kk) == 0
                        want_greater = is_lo if desc_static else (~is_lo)
                    else:
                        desc = ((r * 16 + iota) & kk) == 0
                        want_greater = jnp.logical_not(jnp.logical_xor(desc, is_lo))
                    kp = _perm16(ks[r], perm)
                    ip = _perm16(js[r], perm)
                    g = _cmp_gt(ks[r], js[r], kp, ip)
                    choose_self = want_greater == g
                    ks[r] = jnp.where(choose_self, ks[r], kp)
                    js[r] = jnp.where(choose_self, js[r], ip)
            j //= 2
    return ks, js


def _body(scores_hbm, vals_hbm, idx_hbm, row_v, mid_v, mval_v, mx_v, hist_v,
          tot_v, fin_v, ov_v, oi_v):
    wid = lax.axis_index("s") * 2 + lax.axis_index("c")
    iota = _iota16()
    ones = jnp.ones((16,), jnp.int32)

    def per_row(ri, _):
        row = wid * RPW + ri
        pltpu.sync_copy(scores_hbm.at[row], row_v)

        def pre(i, _):
            m = row_v[pl.ds(i * PRE * 16, 16)]
            for c in range(1, PRE):
                m = jnp.maximum(m, row_v[pl.ds((i * PRE + c) * 16, 16)])
            mx_v[pl.ds(i * 16, 16)] = m
            return 0

        lax.fori_loop(0, NVM, pre, 0, unroll=4)

        _zero_hist(hist_v)

        def h1(i, _):
            u = _ukey(mx_v[pl.ds(i * 16, 16)])
            b = (u >> jnp.uint32(24)).astype(jnp.int32)
            plsc.addupdate_scatter(hist_v, [iota * 256 + b], ones)
            return 0

        lax.fori_loop(0, NVM, h1, 0, unroll=4)
        _totals(hist_v, tot_v)
        b1 = _find_bucket(tot_v, K, iota)
        t_scr = b1.astype(jnp.uint32) << jnp.uint32(24)

        def comp(i, offs):
            new = []
            for q in range(QC):
                u = _ukey(row_v[pl.ds((q * QN + i) * 16, 16)])
                m = u >= t_scr
                plsc.store_compressed(
                    mid_v.at[pl.ds(q * QCAP + offs[q], 16)],
                    (q * QN + i) * 16 + iota, mask=m)
                new.append(offs[q] + _pcount(m))
            return tuple(new)

        nq = lax.fori_loop(0, QN, comp, (0, 0, 0, 0))
        n_mid = nq[0] + nq[1] + nq[2] + nq[3]
        nv_mid = (n_mid + 15) // 16

        cum = nq[0]
        for q in range(1, QC):
            def stitch(j, _, q=q, cum=cum, nqq=nq[q]):
                v = mid_v[pl.ds(q * QCAP + j * 16, 16)]
                plsc.store_compressed(mid_v.at[pl.ds(cum + j * 16, 16)], v,
                                      mask=(j * 16 + iota) < nqq)
                return 0

            lax.fori_loop(0, (nq[q] + 15) // 16, stitch, 0)
            cum = cum + nq[q]

        def cache(i, _):
            idx = mid_v[pl.ds(i * 16, 16)]
            valid = (i * 16 + iota) < n_mid
            mval_v[pl.ds(i * 16, 16)] = plsc.load_gather(
                row_v, [idx], mask=valid)
            return 0

        lax.fori_loop(0, nv_mid, cache, 0)

        n_gt = 0
        prefix = jnp.uint32(0)
        z16 = jnp.zeros((16,), jnp.int32)
        for shift in (28, 24, 20, 16, 12, 8, 4, 0):
            for lane in range(16):
                hist_v[pl.ds(lane * 16, 16)] = z16
            needed = K - n_gt

            def rh(i, _, shift=shift, prefix=prefix):
                u = _ukey(mval_v[pl.ds(i * 16, 16)])
                valid = (i * 16 + iota) < n_mid
                if shift == 28:
                    match = valid
                else:
                    match = jnp.logical_and(
                        valid, (u >> jnp.uint32(shift + 4)) == prefix)
                dig = ((u >> jnp.uint32(shift)) & jnp.uint32(0xF)).astype(
                    jnp.int32)
                plsc.addupdate_scatter(hist_v, [iota * 16 + dig], ones,
                                       mask=match)
                return 0

            lax.fori_loop(0, nv_mid, rh, 0)
            acc = hist_v[pl.ds(0, 16)]
            for lane in range(1, 16):
                acc = acc + hist_v[pl.ds(lane * 16, 16)]
            c = plsc.cumsum(lax.rev(acc, (0,)))
            l_star = _pcount(c < needed)
            bs = 15 - l_star
            cprev = jnp.sum(jnp.where(iota == (l_star - 1), c, 0))
            n_gt = n_gt + cprev
            prefix = (prefix << jnp.uint32(4)) | bs.astype(jnp.uint32)
        t_key = prefix

        def gt_scan(i, off):
            u = _ukey(mval_v[pl.ds(i * 16, 16)])
            valid = (i * 16 + iota) < n_mid
            m = jnp.logical_and(valid, u > t_key)
            plsc.store_compressed(fin_v.at[pl.ds(off, 16)],
                                  mid_v[pl.ds(i * 16, 16)], mask=m)
            return off + _pcount(m)

        off = lax.fori_loop(0, nv_mid, gt_scan, 0)

        def eq_scan(i, off):
            u = _ukey(mval_v[pl.ds(i * 16, 16)])
            valid = (i * 16 + iota) < n_mid
            m = jnp.logical_and(valid, u == t_key)
            pc = _pcount(m)

            @pl.when(off < K)
            def _():
                plsc.store_compressed(fin_v.at[pl.ds(off, 16)],
                                      mid_v[pl.ds(i * 16, 16)], mask=m)

            return jnp.where(off < K, off + pc, off)

        lax.fori_loop(0, nv_mid, eq_scan, off)

        ks, js = [], []
        for r in range(16):
            idxv = fin_v[pl.ds(r * 16, 16)]
            ks.append(_ukey(plsc.load_gather(row_v, [idxv])))
            js.append(idxv)
        ks, js = _bitonic256(ks, js, iota)

        for r in range(16):
            ov_v[pl.ds(r * 16, 16)] = plsc.load_gather(row_v, [js[r]])
            oi_v[pl.ds(r * 16, 16)] = js[r]
        pltpu.sync_copy(ov_v, vals_hbm.at[row])
        pltpu.sync_copy(oi_v, idx_hbm.at[row])
        return 0

    lax.fori_loop(0, RPW, per_row, 0)


@jax.jit
def kernel(scores):
    mesh = plsc.VectorSubcoreMesh(core_axis_name="c", subcore_axis_name="s")
    f = functools.partial(
        pl.kernel,
        mesh=mesh,
        out_type=(
            jax.ShapeDtypeStruct((ROWS, K), jnp.float32),
            jax.ShapeDtypeStruct((ROWS, K), jnp.int32),
        ),
        compiler_params=pltpu.CompilerParams(
            needs_layout_passes=False, disable_bounds_checks=True),
        scratch_types=[
            pltpu.VMEM((N,), jnp.float32),
            pltpu.VMEM((QC * QCAP,), jnp.int32),
            pltpu.VMEM((N + 16,), jnp.float32),
            pltpu.VMEM((N // PRE,), jnp.float32),
            pltpu.VMEM((4096,), jnp.int32),
            pltpu.VMEM((256,), jnp.int32),
            pltpu.VMEM((288,), jnp.int32),
            pltpu.VMEM((K,), jnp.float32),
            pltpu.VMEM((K,), jnp.int32),
        ],
    )(_body)
    return f(scores)
